# baseline (speedup 1.0000x reference)
import functools

import jax
import jax.numpy as jnp
from jax import lax
from jax.experimental import pallas as pl
from jax.experimental.pallas import tpu as pltpu
from jax.experimental.pallas import tpu_sc as plsc

M, K, D = 8, 8192, 256
N, H, W = 16, 32, 32

NC, NS = 2, 16
NW = NC * NS
LANES = 16

B = N * H * W * M
ROWS = B // 128
ROWS_PER_W = ROWS // NW
CHUNK = 128


def _sc_gather(table, codes2):
    mesh = plsc.VectorSubcoreMesh(
        core_axis_name="c", subcore_axis_name="s", num_cores=NC,
        num_subcores=NS)

    @functools.partial(
        pl.kernel,
        mesh=mesh,
        out_type=jax.ShapeDtypeStruct((B, D), jnp.float32),
        scratch_types=[
            pltpu.VMEM((ROWS_PER_W, 128), jnp.int32),
            pltpu.VMEM((CHUNK, D), jnp.float32),
            pltpu.SemaphoreType.DMA,
        ],
    )
    def k(table_hbm, codes_hbm, out_hbm, idx_v, rows_v, sem):
        wid = lax.axis_index("s") * NC + lax.axis_index("c")
        row0 = wid * ROWS_PER_W
        pltpu.sync_copy(codes_hbm.at[pl.ds(row0, ROWS_PER_W)], idx_v)

        mv = (lax.iota(jnp.int32, LANES) % M) * K

        def add_m(j, _):
            for c in range(128 // LANES):
                sl = pl.ds(c * LANES, LANES)
                idx_v[j, sl] = idx_v[j, sl] + mv
            return 0

        lax.fori_loop(0, ROWS_PER_W, add_m, 0)

        def gather_chunk(j, _):
            pltpu.async_copy(table_hbm.at[idx_v.at[j]], rows_v, sem).wait()
            pltpu.sync_copy(
                rows_v, out_hbm.at[pl.ds((row0 + j) * 128, CHUNK)])
            return 0

        lax.fori_loop(0, ROWS_PER_W, gather_chunk, 0)

    return k(table, codes2)


def _tc_transpose(g3):
    T = H * W

    def body(in_ref, out_ref):
        out_ref[...] = jnp.swapaxes(in_ref[...], 1, 2)

    return pl.pallas_call(
        body,
        grid=(N, M),
        in_specs=[pl.BlockSpec((1, T, D), lambda n, m: (n, 0, m))],
        out_specs=pl.BlockSpec((1, D, T), lambda n, m: (n, m, 0)),
        out_shape=jax.ShapeDtypeStruct((N, M * D, T), jnp.float32),
    )(g3)


def kernel(codes, codebook):
    table = codebook.reshape(M * K, D)
    codes2 = codes.reshape(ROWS, 128)
    g = _sc_gather(table, codes2)
    out = _tc_transpose(g.reshape(N, H * W, M * D))
    return out.reshape(N, M * D, H, W)
